# baseline (speedup 1.0000x reference)
import functools

import jax
import jax.numpy as jnp
from jax.experimental import pallas as pl
from jax.experimental.pallas import tpu as pltpu
from jax.experimental.pallas import tpu_sc as plsc

CHUNK = 64
LANES = 16
ROW = 32
CHUNK_WORDS = CHUNK * ROW
NUM_BLOCKS = 128


def _cumsum_chunks(in_vmem, out_vmem, n_chunks):
    @pl.loop(0, n_chunks)
    def _(c):
        base = c * CHUNK_WORDS
        acc0 = in_vmem[pl.ds(base, LANES)]
        acc1 = in_vmem[pl.ds(base + LANES, LANES)]
        out_vmem[pl.ds(base, LANES)] = acc0
        out_vmem[pl.ds(base + LANES, LANES)] = acc1
        for t in range(1, CHUNK):
            off = base + t * ROW
            acc0 = acc0 + in_vmem[pl.ds(off, LANES)]
            out_vmem[pl.ds(off, LANES)] = acc0
            acc1 = acc1 + in_vmem[pl.ds(off + LANES, LANES)]
            out_vmem[pl.ds(off + LANES, LANES)] = acc1


def kernel(g):
    B, T, H = g.shape
    total = B * T * H
    flat = g.reshape(total)
    block_words = total // NUM_BLOCKS
    chunks_per_block = block_words // CHUNK_WORDS

    mesh = plsc.VectorSubcoreMesh(core_axis_name="c", subcore_axis_name="s")

    @functools.partial(
        pl.kernel,
        out_type=jax.ShapeDtypeStruct((total,), jnp.float32),
        mesh=mesh,
    )
    def run(g_hbm, o_hbm):
        def body(in_vmem, out_vmem):
            _cumsum_chunks(in_vmem, out_vmem, chunks_per_block)

        pltpu.emit_pipeline(
            body,
            grid=(NUM_BLOCKS,),
            in_specs=[pl.BlockSpec((block_words,), lambda i: (i,))],
            out_specs=[pl.BlockSpec((block_words,), lambda i: (i,))],
            core_axis_name=("c", "s"),
            dimension_semantics=(pltpu.PARALLEL,),
        )(g_hbm, o_hbm)

    return run(flat).reshape(B, T, H)


# speedup vs baseline: 1.0421x; 1.0421x over previous
import functools

import jax
import jax.numpy as jnp
from jax.experimental import pallas as pl
from jax.experimental.pallas import tpu as pltpu
from jax.experimental.pallas import tpu_sc as plsc

CHUNK = 64
LANES = 16
ROW = 32
CHUNK_WORDS = CHUNK * ROW
NUM_BLOCKS = 128


def _cumsum_chunks(in_vmem, out_vmem, n_chunks):
    @plsc.parallel_loop(0, n_chunks, unroll=n_chunks)
    def _(c):
        base = c * CHUNK_WORDS
        acc0 = in_vmem[pl.ds(base, LANES)]
        acc1 = in_vmem[pl.ds(base + LANES, LANES)]
        out_vmem[pl.ds(base, LANES)] = acc0
        out_vmem[pl.ds(base + LANES, LANES)] = acc1
        for t in range(1, CHUNK):
            off = base + t * ROW
            acc0 = acc0 + in_vmem[pl.ds(off, LANES)]
            out_vmem[pl.ds(off, LANES)] = acc0
            acc1 = acc1 + in_vmem[pl.ds(off + LANES, LANES)]
            out_vmem[pl.ds(off + LANES, LANES)] = acc1


def kernel(g):
    B, T, H = g.shape
    total = B * T * H
    flat = g.reshape(total)
    block_words = total // NUM_BLOCKS
    chunks_per_block = block_words // CHUNK_WORDS

    mesh = plsc.VectorSubcoreMesh(core_axis_name="c", subcore_axis_name="s")

    @functools.partial(
        pl.kernel,
        out_type=jax.ShapeDtypeStruct((total,), jnp.float32),
        mesh=mesh,
    )
    def run(g_hbm, o_hbm):
        def body(in_vmem, out_vmem):
            _cumsum_chunks(in_vmem, out_vmem, chunks_per_block)

        pltpu.emit_pipeline(
            body,
            grid=(NUM_BLOCKS,),
            in_specs=[pl.BlockSpec((block_words,), lambda i: (i,))],
            out_specs=[pl.BlockSpec((block_words,), lambda i: (i,))],
            core_axis_name=("c", "s"),
            dimension_semantics=(pltpu.PARALLEL,),
        )(g_hbm, o_hbm)

    return run(flat).reshape(B, T, H)
